# baseline (speedup 1.0000x reference)
import functools

import jax
import jax.numpy as jnp
from jax import lax
from jax.experimental import pallas as pl
from jax.experimental.pallas import tpu as pltpu
from jax.experimental.pallas import tpu_sc as plsc

NC = 2
NS = 16
NW = NC * NS

SEQ = 200
EMBED = 64
LANES = 16
VPR = EMBED // LANES

SPLITS = ((0, 104), (104, 96))

G = 1
NBUF = 3
TW = 2 * EMBED


def _build(batch):
    seqs_per_w = batch // NW
    ngroups = seqs_per_w // G
    mesh = plsc.VectorSubcoreMesh(core_axis_name="c", subcore_axis_name="s")

    @functools.partial(
        pl.kernel,
        mesh=mesh,
        compiler_params=pltpu.CompilerParams(
            use_tc_tiling_on_sc=False, needs_layout_passes=False),
        out_type=jax.ShapeDtypeStruct((batch, SEQ, 2 * EMBED), jnp.float32),
        scratch_types=[
            pltpu.VMEM((SEQ, EMBED), jnp.float32),
            pltpu.VMEM((NBUF, G, SEQ), jnp.int32),
            pltpu.VMEM((NBUF, G, SEQ, TW), jnp.float32),
            pltpu.SemaphoreType.DMA((NBUF,)),
            pltpu.SemaphoreType.DMA((NBUF,)),
        ],
    )
    def body(x_hbm, tok_hbm, pos_hbm, out_hbm, pos_v, idx_v, rows_v, gsems, osems):
        wid = lax.axis_index("s") * NC + lax.axis_index("c")
        base_seq = wid * seqs_per_w
        pltpu.sync_copy(pos_hbm, pos_v)

        def prefill(b):
            def row(r, c):
                for j in range(VPR):
                    sl = pl.ds(j * LANES, LANES)
                    v = pos_v[r, sl]
                    for s in range(G):
                        rows_v[b, s, r, sl] = v
                return c

            lax.fori_loop(0, SEQ, row, 0, unroll=2)

        def issue_gather(g, b):
            s0 = base_seq + g * G
            pltpu.sync_copy(x_hbm.at[pl.ds(s0, G)], idx_v.at[b])
            for s in range(G):
                for (o, n) in SPLITS:
                    pltpu.async_copy(
                        tok_hbm.at[idx_v.at[b, s, pl.ds(o, n)]],
                        rows_v.at[b, s, pl.ds(o, n)],
                        gsems.at[b], add=True)

        def drain_gather(b):
            for s in range(G):
                for (o, n) in SPLITS:
                    pltpu.make_async_copy(
                        tok_hbm.at[idx_v.at[b, s, pl.ds(o, n)]],
                        rows_v.at[b, s, pl.ds(o, n)],
                        gsems.at[b]).wait()

        def issue_out(g, b):
            s0 = base_seq + g * G
            pltpu.async_copy(rows_v.at[b, :, :, pl.ds(0, EMBED)],
                             out_hbm.at[pl.ds(s0, G), :, pl.ds(0, EMBED)],
                             osems.at[b])

        def drain_out(g, b):
            s0 = base_seq + g * G
            pltpu.make_async_copy(rows_v.at[b, :, :, pl.ds(0, EMBED)],
                                  out_hbm.at[pl.ds(s0, G), :, pl.ds(0, EMBED)],
                                  osems.at[b]).wait()

        prefill(0)
        issue_gather(0, 0)

        def step(g, carry):
            b = lax.rem(g, NBUF)
            bn = lax.rem(g + 1, NBUF)

            @pl.when(g >= 2)
            def _():
                drain_out(g - 2, bn)

            @pl.when(g + 1 < ngroups)
            def _():
                prefill(bn)
                issue_gather(g + 1, bn)

            drain_gather(b)
            issue_out(g, b)
            return carry

        lax.fori_loop(0, ngroups, step, 0)

        for g in (ngroups - 2, ngroups - 1):
            drain_out(g, g % NBUF)

    return body


def kernel(x, token_table, pos_table):
    batch = x.shape[0]
    tok_wide = jnp.pad(token_table, ((0, 0), (0, TW - EMBED)))
    run = _build(batch)
    padded = run(x.astype(jnp.int32), tok_wide, pos_table)
    return padded[:, :, :EMBED]
